# baseline (speedup 1.0000x reference)
import functools

import jax
import jax.numpy as jnp
from jax import lax
from jax.experimental import pallas as pl
from jax.experimental.pallas import tpu as pltpu


def _round_up(a, b):
    return (a + b - 1) // b * b


def _make_body(nf, H, BC, ks, di, pad, *, neg_slope=0.01, eps=1e-5, unroll=8):
    H2, H3 = 2 * H, 3 * H
    H4, H5, H6 = 4 * H, 5 * H, 6 * H
    NR = nf * BC
    f32, bf16 = jnp.float32, jnp.bfloat16

    def body(x_ref, hid_ref, wgi_ref, bgi_ref, whh_ref, bhn_ref,
             wcv_ref, bcv_ref, gnw_ref, gnb_ref, gavg_ref,
             out_ref, rs_ref, c_ref, wgif_s, wgib_s, whh_s, wcv_s):
        wgi = wgi_ref[...]
        wgif_s[...] = jnp.concatenate(
            [wgi[:, 0:H], wgi[:, H2:H3], wgi[:, H4:H5]], axis=1).astype(bf16)
        wgib_s[...] = jnp.concatenate(
            [wgi[:, H:H2], wgi[:, H3:H4], wgi[:, H5:H6]], axis=1).astype(bf16)
        whh = whh_ref[...]
        whh_s[...] = jnp.concatenate(
            [whh[:, 0:H], whh[:, H2:H3], whh[:, H:H2], whh[:, H3:H4],
             whh[:, H4:H5], whh[:, H5:H6]], axis=1).astype(bf16)
        wcv_s[...] = wcv_ref[...].astype(bf16)
        bgi = bgi_ref[...]
        bgif = jnp.concatenate(
            [bgi[:, 0:H], bgi[:, H2:H3], bgi[:, H4:H5]], axis=1)
        bgib = jnp.concatenate(
            [bgi[:, H:H2], bgi[:, H3:H4], bgi[:, H5:H6]], axis=1)
        bhnf = jnp.broadcast_to(bhn_ref[:, 0:H], (BC, H))
        bhnb = jnp.broadcast_to(bhn_ref[:, H:H2], (BC, H))

        rs_ref[...] = jnp.zeros((nf + 2 * pad, BC, H), bf16)

        def step(t, carry):
            hf, hb = carry
            tb = nf - 1 - t
            gf = (jnp.dot(x_ref[t], wgif_s[...], preferred_element_type=f32)
                  + bgif)
            gb = (jnp.dot(x_ref[tb], wgib_s[...], preferred_element_type=f32)
                  + bgib)
            hcat = jnp.concatenate([hf, hb], axis=1).astype(bf16)
            gh_rz = jnp.dot(hcat, whh_s[:, 0:H4], preferred_element_type=f32)
            gh_n = jnp.dot(hcat, whh_s[:, H4:H6], preferred_element_type=f32)
            rzf = 0.5 + 0.5 * jnp.tanh(0.5 * (gf[:, 0:H2] + gh_rz[:, 0:H2]))
            rzb = 0.5 + 0.5 * jnp.tanh(0.5 * (gb[:, 0:H2] + gh_rz[:, H2:H4]))
            nff = jnp.tanh(gf[:, H2:H3] + rzf[:, 0:H] * (gh_n[:, 0:H] + bhnf))
            nbb = jnp.tanh(gb[:, H2:H3] + rzb[:, 0:H] * (gh_n[:, H:H2] + bhnb))
            hf = nff + rzf[:, H:H2] * (hf - nff)
            hb = nbb + rzb[:, H:H2] * (hb - nbb)
            rs_ref[pad + t, :, :] += hf.astype(bf16)
            rs_ref[pad + tb, :, :] += hb.astype(bf16)
            return (hf, hb)

        lax.fori_loop(0, nf, step, (hid_ref[0], hid_ref[1]), unroll=unroll)

        acc = jnp.dot(rs_ref[pl.ds(0, nf), :, :].reshape(NR, H),
                      wcv_s[0:H, :], preferred_element_type=f32)
        for k in range(1, ks):
            acc = acc + jnp.dot(
                rs_ref[pl.ds(k * di, nf), :, :].reshape(NR, H),
                wcv_s[k * H:(k + 1) * H, :], preferred_element_type=f32)
        c_ref[...] = acc.astype(bf16) + bcv_ref[...].astype(bf16)

        inv_nf = 1.0 / float(nf)
        s1 = jnp.sum(c_ref[...].reshape(nf, BC, H).astype(f32), axis=0)
        mean_g = jnp.dot(s1, gavg_ref[...], preferred_element_type=f32) * inv_nf
        c_ref[...] = (c_ref[...].reshape(nf, BC, H)
                      - mean_g[None].astype(bf16)).reshape(NR, H)
        cen3 = c_ref[...].reshape(nf, BC, H).astype(f32)
        s2 = jnp.sum(cen3 * cen3, axis=0)
        var_g = jnp.dot(s2, gavg_ref[...], preferred_element_type=f32) * inv_nf
        inv = lax.rsqrt(var_g + eps)
        cn3 = cen3 * inv[None] * gnw_ref[...] + gnb_ref[...]
        cact = jnp.where(cn3 >= 0, cn3, neg_slope * cn3)
        rsum = rs_ref[pl.ds(pad, nf), :, :].astype(f32)
        out_ref[...] = x_ref[...].astype(f32) + rsum + cact

    return body


def kernel(x, hidden, w_gi, b_gi, w_hh, b_hn, w_cv, b_cv, gn_w, gn_b, gavg):
    bs, mfd, nf = x.shape
    H = mfd
    ks = w_cv.shape[0] // H
    di = 2
    pad = (ks - 1) * di // 2
    H2, H3 = 2 * H, 3 * H
    f32, bf16 = jnp.float32, jnp.bfloat16

    BC = min(128, _round_up(bs, 8))
    bsp = _round_up(bs, BC)
    nchunk = bsp // BC

    xb = jnp.transpose(x, (2, 0, 1)).astype(bf16)
    hid = hidden
    if bsp != bs:
        xb = jnp.pad(xb, ((0, 0), (0, bsp - bs), (0, 0)))
        hid = jnp.pad(hidden, ((0, 0), (0, bsp - bs), (0, 0)))

    unroll = 1
    for cand in (16, 8, 4, 2):
        if nf % cand == 0:
            unroll = cand
            break

    body = _make_body(nf, H, BC, ks, di, pad, unroll=unroll)
    full = lambda shape: pl.BlockSpec(shape, lambda i: (0,) * len(shape))

    out_t = pl.pallas_call(
        body,
        out_shape=jax.ShapeDtypeStruct((nf, bsp, H), f32),
        grid=(nchunk,),
        in_specs=[
            pl.BlockSpec((nf, BC, H), lambda i: (0, i, 0)),
            pl.BlockSpec((2, BC, H), lambda i: (0, i, 0)),
            full((H, 6 * H)), full((1, 6 * H)),
            full((H2, 6 * H)), full((1, H2)),
            full((ks * H, H)), full((1, H)),
            full((1, H)), full((1, H)),
            full((H, H)),
        ],
        out_specs=pl.BlockSpec((nf, BC, H), lambda i: (0, i, 0)),
        scratch_shapes=[
            pltpu.VMEM((nf + 2 * pad, BC, H), bf16),
            pltpu.VMEM((nf * BC, H), bf16),
            pltpu.VMEM((H, H3), bf16),
            pltpu.VMEM((H, H3), bf16),
            pltpu.VMEM((H2, 6 * H), bf16),
            pltpu.VMEM((ks * H, H), bf16),
        ],
        compiler_params=pltpu.CompilerParams(
            dimension_semantics=("parallel",),
            vmem_limit_bytes=62 * 1024 * 1024),
    )(xb, hid, w_gi, b_gi, w_hh, b_hn, w_cv, b_cv, gn_w, gn_b, gavg)

    return jnp.transpose(out_t[:, :bs, :], (1, 2, 0))


# speedup vs baseline: 1.0018x; 1.0018x over previous
import functools

import jax
import jax.numpy as jnp
from jax import lax
from jax.experimental import pallas as pl
from jax.experimental.pallas import tpu as pltpu


def _round_up(a, b):
    return (a + b - 1) // b * b


def _make_body(nf, H, BC, ks, di, pad, *, neg_slope=0.01, eps=1e-5, unroll=8):
    H2, H3 = 2 * H, 3 * H
    H4, H5, H6 = 4 * H, 5 * H, 6 * H
    NR = nf * BC
    f32, bf16 = jnp.float32, jnp.bfloat16

    def body(x_ref, hid_ref, wgi_ref, bgi_ref, whh_ref, bhn_ref,
             wcv_ref, bcv_ref, gnw_ref, gnb_ref, gavg_ref,
             out_ref, rs_ref, c_ref, wgif_s, wgib_s, whh_s, wcv_s):
        wgi = wgi_ref[...]
        wgif_s[...] = jnp.concatenate(
            [wgi[:, 0:H], wgi[:, H2:H3], wgi[:, H4:H5]], axis=1).astype(bf16)
        wgib_s[...] = jnp.concatenate(
            [wgi[:, H:H2], wgi[:, H3:H4], wgi[:, H5:H6]], axis=1).astype(bf16)
        whh = whh_ref[...]
        whh_s[...] = jnp.concatenate(
            [whh[:, 0:H], whh[:, H2:H3], whh[:, H:H2], whh[:, H3:H4],
             whh[:, H4:H5], whh[:, H5:H6]], axis=1).astype(bf16)
        wcv_s[...] = wcv_ref[...].astype(bf16)
        bgi = bgi_ref[...]
        bgif = jnp.concatenate(
            [bgi[:, 0:H], bgi[:, H2:H3], bgi[:, H4:H5]], axis=1)
        bgib = jnp.concatenate(
            [bgi[:, H:H2], bgi[:, H3:H4], bgi[:, H5:H6]], axis=1)
        bhnf = jnp.broadcast_to(bhn_ref[:, 0:H], (BC, H))
        bhnb = jnp.broadcast_to(bhn_ref[:, H:H2], (BC, H))

        rs_ref[...] = jnp.zeros((nf + 2 * pad, BC, H), bf16)

        def step(t, carry):
            hf, hb = carry
            tb = nf - 1 - t
            gf = (jnp.dot(x_ref[t], wgif_s[...], preferred_element_type=f32)
                  + bgif)
            gb = (jnp.dot(x_ref[tb], wgib_s[...], preferred_element_type=f32)
                  + bgib)
            hcat = jnp.concatenate([hf, hb], axis=1).astype(bf16)
            gh = jnp.dot(hcat, whh_s[...], preferred_element_type=f32)
            rzf = 0.5 + 0.5 * jnp.tanh(0.5 * (gf[:, 0:H2] + gh[:, 0:H2]))
            rzb = 0.5 + 0.5 * jnp.tanh(0.5 * (gb[:, 0:H2] + gh[:, H2:H4]))
            nff = jnp.tanh(gf[:, H2:H3] + rzf[:, 0:H] * (gh[:, H4:H5] + bhnf))
            nbb = jnp.tanh(gb[:, H2:H3] + rzb[:, 0:H] * (gh[:, H5:H6] + bhnb))
            hf = nff + rzf[:, H:H2] * (hf - nff)
            hb = nbb + rzb[:, H:H2] * (hb - nbb)
            rs_ref[pad + t, :, :] += hf.astype(bf16)
            rs_ref[pad + tb, :, :] += hb.astype(bf16)
            return (hf, hb)

        lax.fori_loop(0, nf, step, (hid_ref[0], hid_ref[1]), unroll=unroll)

        acc = jnp.dot(rs_ref[pl.ds(0, nf), :, :].reshape(NR, H),
                      wcv_s[0:H, :], preferred_element_type=f32)
        for k in range(1, ks):
            acc = acc + jnp.dot(
                rs_ref[pl.ds(k * di, nf), :, :].reshape(NR, H),
                wcv_s[k * H:(k + 1) * H, :], preferred_element_type=f32)
        c_ref[...] = acc.astype(bf16) + bcv_ref[...].astype(bf16)

        inv_nf = 1.0 / float(nf)
        s1 = jnp.sum(c_ref[...].reshape(nf, BC, H).astype(f32), axis=0)
        mean_g = jnp.dot(s1, gavg_ref[...], preferred_element_type=f32) * inv_nf
        c_ref[...] = (c_ref[...].reshape(nf, BC, H)
                      - mean_g[None].astype(bf16)).reshape(NR, H)
        cen3 = c_ref[...].reshape(nf, BC, H).astype(f32)
        s2 = jnp.sum(cen3 * cen3, axis=0)
        var_g = jnp.dot(s2, gavg_ref[...], preferred_element_type=f32) * inv_nf
        inv = lax.rsqrt(var_g + eps)
        cn3 = cen3 * inv[None] * gnw_ref[...] + gnb_ref[...]
        cact = jnp.where(cn3 >= 0, cn3, neg_slope * cn3)
        rsum = rs_ref[pl.ds(pad, nf), :, :].astype(f32)
        out_ref[...] = x_ref[...].astype(f32) + rsum + cact

    return body


def kernel(x, hidden, w_gi, b_gi, w_hh, b_hn, w_cv, b_cv, gn_w, gn_b, gavg):
    bs, mfd, nf = x.shape
    H = mfd
    ks = w_cv.shape[0] // H
    di = 2
    pad = (ks - 1) * di // 2
    H2, H3 = 2 * H, 3 * H
    f32, bf16 = jnp.float32, jnp.bfloat16

    BC = min(128, _round_up(bs, 8))
    bsp = _round_up(bs, BC)
    nchunk = bsp // BC

    xb = jnp.transpose(x, (2, 0, 1)).astype(bf16)
    hid = hidden
    if bsp != bs:
        xb = jnp.pad(xb, ((0, 0), (0, bsp - bs), (0, 0)))
        hid = jnp.pad(hidden, ((0, 0), (0, bsp - bs), (0, 0)))

    unroll = 1
    for cand in (16, 8, 4, 2):
        if nf % cand == 0:
            unroll = cand
            break

    body = _make_body(nf, H, BC, ks, di, pad, unroll=unroll)
    full = lambda shape: pl.BlockSpec(shape, lambda i: (0,) * len(shape))

    out_t = pl.pallas_call(
        body,
        out_shape=jax.ShapeDtypeStruct((nf, bsp, H), f32),
        grid=(nchunk,),
        in_specs=[
            pl.BlockSpec((nf, BC, H), lambda i: (0, i, 0)),
            pl.BlockSpec((2, BC, H), lambda i: (0, i, 0)),
            full((H, 6 * H)), full((1, 6 * H)),
            full((H2, 6 * H)), full((1, H2)),
            full((ks * H, H)), full((1, H)),
            full((1, H)), full((1, H)),
            full((H, H)),
        ],
        out_specs=pl.BlockSpec((nf, BC, H), lambda i: (0, i, 0)),
        scratch_shapes=[
            pltpu.VMEM((nf + 2 * pad, BC, H), bf16),
            pltpu.VMEM((nf * BC, H), bf16),
            pltpu.VMEM((H, H3), bf16),
            pltpu.VMEM((H, H3), bf16),
            pltpu.VMEM((H2, 6 * H), bf16),
            pltpu.VMEM((ks * H, H), bf16),
        ],
        compiler_params=pltpu.CompilerParams(
            dimension_semantics=("parallel",),
            vmem_limit_bytes=62 * 1024 * 1024),
    )(xb, hid, w_gi, b_gi, w_hh, b_hn, w_cv, b_cv, gn_w, gn_b, gavg)

    return jnp.transpose(out_t[:, :bs, :], (1, 2, 0))


# speedup vs baseline: 1.0154x; 1.0136x over previous
import functools

import jax
import jax.numpy as jnp
from jax import lax
from jax.experimental import pallas as pl
from jax.experimental.pallas import tpu as pltpu


def _round_up(a, b):
    return (a + b - 1) // b * b


def _make_body(nf, H, BC, ks, di, pad, *, neg_slope=0.01, eps=1e-5, unroll=8):
    H2, H3 = 2 * H, 3 * H
    H4, H5, H6 = 4 * H, 5 * H, 6 * H
    NR = nf * BC
    f32, bf16 = jnp.float32, jnp.bfloat16

    def body(x_ref, hid_ref, wgi_ref, bgi_ref, whh_ref, bhn_ref,
             wcv_ref, bcv_ref, gnw_ref, gnb_ref, gavg_ref,
             out_ref, rs_ref, c_ref, wgif_s, wgib_s, whh_s, wcv_s):
        wgi = wgi_ref[...]
        wgif_s[...] = jnp.concatenate(
            [wgi[:, 0:H], wgi[:, H2:H3], wgi[:, H4:H5]], axis=1).astype(bf16)
        wgib_s[...] = jnp.concatenate(
            [wgi[:, H:H2], wgi[:, H3:H4], wgi[:, H5:H6]], axis=1).astype(bf16)
        whh = whh_ref[...]
        whh_s[...] = jnp.concatenate(
            [whh[:, 0:H], whh[:, H2:H3], whh[:, H:H2], whh[:, H3:H4],
             whh[:, H4:H5], whh[:, H5:H6]], axis=1).astype(bf16)
        wcv_s[...] = wcv_ref[...].astype(bf16)
        bgi = bgi_ref[...]
        bgif = jnp.concatenate(
            [bgi[:, 0:H], bgi[:, H2:H3], bgi[:, H4:H5]], axis=1)
        bgib = jnp.concatenate(
            [bgi[:, H:H2], bgi[:, H3:H4], bgi[:, H5:H6]], axis=1)
        bhnf = jnp.broadcast_to(bhn_ref[:, 0:H], (BC, H))
        bhnb = jnp.broadcast_to(bhn_ref[:, H:H2], (BC, H))

        rs_ref[...] = jnp.zeros((nf + 2 * pad, BC, H), bf16)

        def step(t, carry):
            hf, hb = carry
            tb = nf - 1 - t
            gf = (jnp.dot(x_ref[t], wgif_s[...], preferred_element_type=f32)
                  + bgif)
            gb = (jnp.dot(x_ref[tb], wgib_s[...], preferred_element_type=f32)
                  + bgib)
            hcat = jnp.concatenate([hf, hb], axis=1).astype(bf16)
            gh = jnp.dot(hcat, whh_s[...], preferred_element_type=f32)
            rzf = jax.nn.sigmoid(gf[:, 0:H2] + gh[:, 0:H2])
            rzb = jax.nn.sigmoid(gb[:, 0:H2] + gh[:, H2:H4])
            nff = jnp.tanh(gf[:, H2:H3] + rzf[:, 0:H] * (gh[:, H4:H5] + bhnf))
            nbb = jnp.tanh(gb[:, H2:H3] + rzb[:, 0:H] * (gh[:, H5:H6] + bhnb))
            hf = nff + rzf[:, H:H2] * (hf - nff)
            hb = nbb + rzb[:, H:H2] * (hb - nbb)
            rs_ref[pad + t, :, :] += hf.astype(bf16)
            rs_ref[pad + tb, :, :] += hb.astype(bf16)
            return (hf, hb)

        lax.fori_loop(0, nf, step, (hid_ref[0], hid_ref[1]), unroll=unroll)

        acc = jnp.dot(rs_ref[pl.ds(0, nf), :, :].reshape(NR, H),
                      wcv_s[0:H, :], preferred_element_type=f32)
        for k in range(1, ks):
            acc = acc + jnp.dot(
                rs_ref[pl.ds(k * di, nf), :, :].reshape(NR, H),
                wcv_s[k * H:(k + 1) * H, :], preferred_element_type=f32)
        c_ref[...] = acc.astype(bf16) + bcv_ref[...].astype(bf16)

        inv_nf = 1.0 / float(nf)
        s1 = jnp.sum(c_ref[...].reshape(nf, BC, H).astype(f32), axis=0)
        mean_g = jnp.dot(s1, gavg_ref[...], preferred_element_type=f32) * inv_nf
        c_ref[...] = (c_ref[...].reshape(nf, BC, H)
                      - mean_g[None].astype(bf16)).reshape(NR, H)
        cen3 = c_ref[...].reshape(nf, BC, H).astype(f32)
        s2 = jnp.sum(cen3 * cen3, axis=0)
        var_g = jnp.dot(s2, gavg_ref[...], preferred_element_type=f32) * inv_nf
        inv = lax.rsqrt(var_g + eps)
        cn3 = cen3 * inv[None] * gnw_ref[...] + gnb_ref[...]
        cact = jnp.where(cn3 >= 0, cn3, neg_slope * cn3)
        rsum = rs_ref[pl.ds(pad, nf), :, :].astype(f32)
        out_ref[...] = x_ref[...].astype(f32) + rsum + cact

    return body


def kernel(x, hidden, w_gi, b_gi, w_hh, b_hn, w_cv, b_cv, gn_w, gn_b, gavg):
    bs, mfd, nf = x.shape
    H = mfd
    ks = w_cv.shape[0] // H
    di = 2
    pad = (ks - 1) * di // 2
    H2, H3 = 2 * H, 3 * H
    f32, bf16 = jnp.float32, jnp.bfloat16

    BC = min(128, _round_up(bs, 8))
    bsp = _round_up(bs, BC)
    nchunk = bsp // BC

    xb = jnp.transpose(x, (2, 0, 1)).astype(bf16)
    hid = hidden
    if bsp != bs:
        xb = jnp.pad(xb, ((0, 0), (0, bsp - bs), (0, 0)))
        hid = jnp.pad(hidden, ((0, 0), (0, bsp - bs), (0, 0)))

    unroll = 1
    for cand in (16, 8, 4, 2):
        if nf % cand == 0:
            unroll = cand
            break

    body = _make_body(nf, H, BC, ks, di, pad, unroll=unroll)
    full = lambda shape: pl.BlockSpec(shape, lambda i: (0,) * len(shape))

    out_t = pl.pallas_call(
        body,
        out_shape=jax.ShapeDtypeStruct((nf, bsp, H), f32),
        grid=(nchunk,),
        in_specs=[
            pl.BlockSpec((nf, BC, H), lambda i: (0, i, 0)),
            pl.BlockSpec((2, BC, H), lambda i: (0, i, 0)),
            full((H, 6 * H)), full((1, 6 * H)),
            full((H2, 6 * H)), full((1, H2)),
            full((ks * H, H)), full((1, H)),
            full((1, H)), full((1, H)),
            full((H, H)),
        ],
        out_specs=pl.BlockSpec((nf, BC, H), lambda i: (0, i, 0)),
        scratch_shapes=[
            pltpu.VMEM((nf + 2 * pad, BC, H), bf16),
            pltpu.VMEM((nf * BC, H), bf16),
            pltpu.VMEM((H, H3), bf16),
            pltpu.VMEM((H, H3), bf16),
            pltpu.VMEM((H2, 6 * H), bf16),
            pltpu.VMEM((ks * H, H), bf16),
        ],
        compiler_params=pltpu.CompilerParams(
            dimension_semantics=("parallel",),
            vmem_limit_bytes=62 * 1024 * 1024),
    )(xb, hid, w_gi, b_gi, w_hh, b_hn, w_cv, b_cv, gn_w, gn_b, gavg)

    return jnp.transpose(out_t[:, :bs, :], (1, 2, 0))


# speedup vs baseline: 1.1053x; 1.0885x over previous
import functools

import jax
import jax.numpy as jnp
from jax import lax
from jax.experimental import pallas as pl
from jax.experimental.pallas import tpu as pltpu


def _round_up(a, b):
    return (a + b - 1) // b * b


def _make_body(nf, H, BC, ks, di, pad, *, neg_slope=0.01, eps=1e-5, unroll=8):
    H2, H3 = 2 * H, 3 * H
    H4, H5, H6 = 4 * H, 5 * H, 6 * H
    NR = nf * BC
    f32, bf16 = jnp.float32, jnp.bfloat16

    def body(x_ref, hid_ref, wgi_ref, bgi_ref, whh_ref, bhn_ref,
             wcv_ref, bcv_ref, gnw_ref, gnb_ref, gavg_ref,
             out_ref, rs_ref, c_ref, wgif_s, wgib_s, whh_s, wcv_s):
        wgi = wgi_ref[...]
        wgif_s[...] = jnp.concatenate(
            [wgi[:, 0:H], wgi[:, H2:H3], wgi[:, H4:H5]], axis=1).astype(bf16)
        wgib_s[...] = jnp.concatenate(
            [wgi[:, H:H2], wgi[:, H3:H4], wgi[:, H5:H6]], axis=1).astype(bf16)
        whh = whh_ref[...]
        whh_s[...] = jnp.concatenate(
            [whh[:, 0:H], whh[:, H2:H3], whh[:, H4:H5],
             whh[:, H:H2], whh[:, H3:H4], whh[:, H5:H6]], axis=1).astype(bf16)
        wcv_s[...] = wcv_ref[...].astype(bf16)
        bgi = bgi_ref[...]
        bgif = jnp.concatenate(
            [bgi[:, 0:H], bgi[:, H2:H3], bgi[:, H4:H5]], axis=1)
        bgib = jnp.concatenate(
            [bgi[:, H:H2], bgi[:, H3:H4], bgi[:, H5:H6]], axis=1)
        bhnf = jnp.broadcast_to(bhn_ref[:, 0:H], (BC, H))
        bhnb = jnp.broadcast_to(bhn_ref[:, H:H2], (BC, H))

        rs_ref[...] = jnp.zeros((nf + 2 * pad, BC, H), bf16)

        def step(t, carry):
            hf, hb = carry
            tb = nf - 1 - t
            gf = (jnp.dot(x_ref[t], wgif_s[...], preferred_element_type=f32)
                  + bgif)
            gb = (jnp.dot(x_ref[tb], wgib_s[...], preferred_element_type=f32)
                  + bgib)
            hcat = jnp.concatenate([hf, hb], axis=1).astype(bf16)
            gh = jnp.dot(hcat, whh_s[...], preferred_element_type=f32)
            ghf = gh[:, 0:H3]
            ghb = gh[:, H3:H6]
            rzf = jax.nn.sigmoid(gf[:, 0:H2] + ghf[:, 0:H2])
            rzb = jax.nn.sigmoid(gb[:, 0:H2] + ghb[:, 0:H2])
            nff = jnp.tanh(gf[:, H2:H3] + rzf[:, 0:H] * (ghf[:, H2:H3] + bhnf))
            nbb = jnp.tanh(gb[:, H2:H3] + rzb[:, 0:H] * (ghb[:, H2:H3] + bhnb))
            hf = nff + rzf[:, H:H2] * (hf - nff)
            hb = nbb + rzb[:, H:H2] * (hb - nbb)
            rs_ref[pad + t, :, :] += hf.astype(bf16)
            rs_ref[pad + tb, :, :] += hb.astype(bf16)
            return (hf, hb)

        lax.fori_loop(0, nf, step, (hid_ref[0], hid_ref[1]), unroll=unroll)

        acc = jnp.dot(rs_ref[pl.ds(0, nf), :, :].reshape(NR, H),
                      wcv_s[0:H, :], preferred_element_type=f32)
        for k in range(1, ks):
            acc = acc + jnp.dot(
                rs_ref[pl.ds(k * di, nf), :, :].reshape(NR, H),
                wcv_s[k * H:(k + 1) * H, :], preferred_element_type=f32)
        c_ref[...] = acc.astype(bf16) + bcv_ref[...].astype(bf16)

        inv_nf = 1.0 / float(nf)
        s1 = jnp.sum(c_ref[...].reshape(nf, BC, H).astype(f32), axis=0)
        mean_g = jnp.dot(s1, gavg_ref[...], preferred_element_type=f32) * inv_nf
        c_ref[...] = (c_ref[...].reshape(nf, BC, H)
                      - mean_g[None].astype(bf16)).reshape(NR, H)
        cen3 = c_ref[...].reshape(nf, BC, H).astype(f32)
        s2 = jnp.sum(cen3 * cen3, axis=0)
        var_g = jnp.dot(s2, gavg_ref[...], preferred_element_type=f32) * inv_nf
        inv = lax.rsqrt(var_g + eps)
        cn3 = cen3 * inv[None] * gnw_ref[...] + gnb_ref[...]
        cact = jnp.where(cn3 >= 0, cn3, neg_slope * cn3)
        rsum = rs_ref[pl.ds(pad, nf), :, :].astype(f32)
        out_ref[...] = x_ref[...].astype(f32) + rsum + cact

    return body


def kernel(x, hidden, w_gi, b_gi, w_hh, b_hn, w_cv, b_cv, gn_w, gn_b, gavg):
    bs, mfd, nf = x.shape
    H = mfd
    ks = w_cv.shape[0] // H
    di = 2
    pad = (ks - 1) * di // 2
    H2, H3 = 2 * H, 3 * H
    f32, bf16 = jnp.float32, jnp.bfloat16

    BC = min(128, _round_up(bs, 8))
    bsp = _round_up(bs, BC)
    nchunk = bsp // BC

    xb = jnp.transpose(x, (2, 0, 1)).astype(bf16)
    hid = hidden
    if bsp != bs:
        xb = jnp.pad(xb, ((0, 0), (0, bsp - bs), (0, 0)))
        hid = jnp.pad(hidden, ((0, 0), (0, bsp - bs), (0, 0)))

    unroll = 1
    for cand in (16, 8, 4, 2):
        if nf % cand == 0:
            unroll = cand
            break

    body = _make_body(nf, H, BC, ks, di, pad, unroll=unroll)
    full = lambda shape: pl.BlockSpec(shape, lambda i: (0,) * len(shape))

    out_t = pl.pallas_call(
        body,
        out_shape=jax.ShapeDtypeStruct((nf, bsp, H), f32),
        grid=(nchunk,),
        in_specs=[
            pl.BlockSpec((nf, BC, H), lambda i: (0, i, 0)),
            pl.BlockSpec((2, BC, H), lambda i: (0, i, 0)),
            full((H, 6 * H)), full((1, 6 * H)),
            full((H2, 6 * H)), full((1, H2)),
            full((ks * H, H)), full((1, H)),
            full((1, H)), full((1, H)),
            full((H, H)),
        ],
        out_specs=pl.BlockSpec((nf, BC, H), lambda i: (0, i, 0)),
        scratch_shapes=[
            pltpu.VMEM((nf + 2 * pad, BC, H), bf16),
            pltpu.VMEM((nf * BC, H), bf16),
            pltpu.VMEM((H, H3), bf16),
            pltpu.VMEM((H, H3), bf16),
            pltpu.VMEM((H2, 6 * H), bf16),
            pltpu.VMEM((ks * H, H), bf16),
        ],
        compiler_params=pltpu.CompilerParams(
            dimension_semantics=("parallel",),
            vmem_limit_bytes=62 * 1024 * 1024),
    )(xb, hid, w_gi, b_gi, w_hh, b_hn, w_cv, b_cv, gn_w, gn_b, gavg)

    return jnp.transpose(out_t[:, :bs, :], (1, 2, 0))


# speedup vs baseline: 1.1134x; 1.0074x over previous
import functools

import jax
import jax.numpy as jnp
from jax import lax
from jax.experimental import pallas as pl
from jax.experimental.pallas import tpu as pltpu


def _round_up(a, b):
    return (a + b - 1) // b * b


def _make_body(nf, H, BC, ks, di, pad, *, neg_slope=0.01, eps=1e-5, unroll=8):
    H2, H3 = 2 * H, 3 * H
    H4, H5, H6 = 4 * H, 5 * H, 6 * H
    NR = nf * BC
    f32, bf16 = jnp.float32, jnp.bfloat16

    def body(x_ref, hid_ref, wgi_ref, bgi_ref, whh_ref, bhn_ref,
             wcv_ref, bcv_ref, gnw_ref, gnb_ref, gavg_ref,
             out_ref, rs_ref, c_ref, wgif_s, wgib_s, whh_s, wcv_s):
        wgi = wgi_ref[...]
        wgif_s[...] = jnp.concatenate(
            [wgi[:, 0:H], wgi[:, H2:H3], wgi[:, H4:H5]], axis=1).astype(bf16)
        wgib_s[...] = jnp.concatenate(
            [wgi[:, H:H2], wgi[:, H3:H4], wgi[:, H5:H6]], axis=1).astype(bf16)
        whh = whh_ref[...]
        whh_s[...] = jnp.concatenate(
            [whh[:, 0:H], whh[:, H2:H3], whh[:, H4:H5],
             whh[:, H:H2], whh[:, H3:H4], whh[:, H5:H6]], axis=1).astype(bf16)
        wcv_s[...] = wcv_ref[...].astype(bf16)
        bgi = bgi_ref[...]
        bgif = jnp.concatenate(
            [bgi[:, 0:H], bgi[:, H2:H3], bgi[:, H4:H5]], axis=1)
        bgib = jnp.concatenate(
            [bgi[:, H:H2], bgi[:, H3:H4], bgi[:, H5:H6]], axis=1)
        bhnf = jnp.broadcast_to(bhn_ref[:, 0:H], (BC, H))
        bhnb = jnp.broadcast_to(bhn_ref[:, H:H2], (BC, H))

        rs_ref[...] = jnp.zeros((nf + 2 * pad, BC, H), bf16)

        def step(t, carry):
            hf, hb = carry
            tb = nf - 1 - t
            gf = (jnp.dot(x_ref[t], wgif_s[...], preferred_element_type=f32)
                  + bgif)
            gb = (jnp.dot(x_ref[tb], wgib_s[...], preferred_element_type=f32)
                  + bgib)
            hcat = jnp.concatenate([hf, hb], axis=1).astype(bf16)
            gh = jnp.dot(hcat, whh_s[...], preferred_element_type=f32)
            ghf = gh[:, 0:H3]
            ghb = gh[:, H3:H6]
            rzf = jax.nn.sigmoid(gf[:, 0:H2] + ghf[:, 0:H2])
            rzb = jax.nn.sigmoid(gb[:, 0:H2] + ghb[:, 0:H2])
            nff = jnp.tanh(gf[:, H2:H3] + rzf[:, 0:H] * (ghf[:, H2:H3] + bhnf))
            nbb = jnp.tanh(gb[:, H2:H3] + rzb[:, 0:H] * (ghb[:, H2:H3] + bhnb))
            hf = nff + rzf[:, H:H2] * (hf - nff)
            hb = nbb + rzb[:, H:H2] * (hb - nbb)
            rs_ref[pad + t, :, :] += hf.astype(bf16)
            rs_ref[pad + tb, :, :] += hb.astype(bf16)
            return (hf, hb)

        lax.fori_loop(0, nf, step, (hid_ref[0], hid_ref[1]), unroll=unroll)

        acc = jnp.dot(rs_ref[pl.ds(0, nf), :, :].reshape(NR, H),
                      wcv_s[0:H, :], preferred_element_type=f32)
        for k in range(1, ks):
            acc = acc + jnp.dot(
                rs_ref[pl.ds(k * di, nf), :, :].reshape(NR, H),
                wcv_s[k * H:(k + 1) * H, :], preferred_element_type=f32)
        c_ref[...] = acc.astype(bf16) + bcv_ref[...].astype(bf16)

        inv_nf = 1.0 / float(nf)
        s1 = jnp.sum(c_ref[...].reshape(nf, BC, H).astype(f32), axis=0)
        mean_g = jnp.dot(s1, gavg_ref[...], preferred_element_type=f32) * inv_nf
        c_ref[...] = (c_ref[...].reshape(nf, BC, H)
                      - mean_g[None].astype(bf16)).reshape(NR, H)
        cen3 = c_ref[...].reshape(nf, BC, H).astype(f32)
        s2 = jnp.sum(cen3 * cen3, axis=0)
        var_g = jnp.dot(s2, gavg_ref[...], preferred_element_type=f32) * inv_nf
        inv = lax.rsqrt(var_g + eps)
        cn3 = cen3 * inv[None] * gnw_ref[...] + gnb_ref[...]
        cact = jnp.where(cn3 >= 0, cn3, neg_slope * cn3)
        rsum = rs_ref[pl.ds(pad, nf), :, :].astype(f32)
        out_ref[...] = x_ref[...].astype(f32) + rsum + cact

    return body


def kernel(x, hidden, w_gi, b_gi, w_hh, b_hn, w_cv, b_cv, gn_w, gn_b, gavg):
    bs, mfd, nf = x.shape
    H = mfd
    ks = w_cv.shape[0] // H
    di = 2
    pad = (ks - 1) * di // 2
    H2, H3 = 2 * H, 3 * H
    f32, bf16 = jnp.float32, jnp.bfloat16

    BC = min(128, _round_up(bs, 8))
    bsp = _round_up(bs, BC)
    nchunk = bsp // BC

    xb = jnp.transpose(x, (2, 0, 1)).astype(bf16)
    hid = hidden
    if bsp != bs:
        xb = jnp.pad(xb, ((0, 0), (0, bsp - bs), (0, 0)))
        hid = jnp.pad(hidden, ((0, 0), (0, bsp - bs), (0, 0)))

    unroll = 1
    for cand in (32, 16, 8, 4, 2):
        if nf % cand == 0:
            unroll = cand
            break

    body = _make_body(nf, H, BC, ks, di, pad, unroll=unroll)
    full = lambda shape: pl.BlockSpec(shape, lambda i: (0,) * len(shape))

    out_t = pl.pallas_call(
        body,
        out_shape=jax.ShapeDtypeStruct((nf, bsp, H), f32),
        grid=(nchunk,),
        in_specs=[
            pl.BlockSpec((nf, BC, H), lambda i: (0, i, 0)),
            pl.BlockSpec((2, BC, H), lambda i: (0, i, 0)),
            full((H, 6 * H)), full((1, 6 * H)),
            full((H2, 6 * H)), full((1, H2)),
            full((ks * H, H)), full((1, H)),
            full((1, H)), full((1, H)),
            full((H, H)),
        ],
        out_specs=pl.BlockSpec((nf, BC, H), lambda i: (0, i, 0)),
        scratch_shapes=[
            pltpu.VMEM((nf + 2 * pad, BC, H), bf16),
            pltpu.VMEM((nf * BC, H), bf16),
            pltpu.VMEM((H, H3), bf16),
            pltpu.VMEM((H, H3), bf16),
            pltpu.VMEM((H2, 6 * H), bf16),
            pltpu.VMEM((ks * H, H), bf16),
        ],
        compiler_params=pltpu.CompilerParams(
            dimension_semantics=("parallel",),
            vmem_limit_bytes=62 * 1024 * 1024),
    )(xb, hid, w_gi, b_gi, w_hh, b_hn, w_cv, b_cv, gn_w, gn_b, gavg)

    return jnp.transpose(out_t[:, :bs, :], (1, 2, 0))
